# baseline (speedup 1.0000x reference)
import functools

import jax
import jax.numpy as jnp
from jax import lax
from jax.experimental import pallas as pl
from jax.experimental.pallas import tpu as pltpu
from jax.experimental.pallas import tpu_sc as plsc

NC, NS = 2, 16
NW = NC * NS
LANES = 16
BLK = 120


def _make_sc_call(R, D, rows_per_w):
    nblk = rows_per_w // BLK
    mesh = plsc.VectorSubcoreMesh(core_axis_name="c", subcore_axis_name="s")

    @functools.partial(
        pl.kernel,
        out_type=jax.ShapeDtypeStruct((R, D), jnp.float32),
        mesh=mesh,
        scratch_types=[
            pltpu.VMEM((BLK,), jnp.int32),
            pltpu.VMEM((BLK,), jnp.int32),
            pltpu.VMEM((BLK,), jnp.int32),
            pltpu.VMEM((BLK, D), jnp.float32),
            pltpu.VMEM((BLK, D), jnp.float32),
            pltpu.VMEM((BLK, D), jnp.float32),
            pltpu.VMEM((BLK, D), jnp.float32),
            pltpu.SemaphoreType.DMA,
        ],
        compiler_params=pltpu.CompilerParams(use_tc_tiling_on_sc=False),
    )
    def sc_call(xf, hidx, didx, sidx, tt, dt, st, out,
                hib, dib, sib, xb, tb, db, sb, sem):
        wid = lax.axis_index("s") * NC + lax.axis_index("c")
        row0 = wid * rows_per_w

        def block_body(k, carry):
            r0 = row0 + k * BLK
            rows = pl.ds(r0, BLK)
            pltpu.sync_copy(hidx.at[rows], hib)
            pltpu.sync_copy(didx.at[rows], dib)
            pltpu.sync_copy(sidx.at[rows], sib)
            cx = pltpu.async_copy(xf.at[rows], xb, sem)
            ct = pltpu.async_copy(tt.at[hib], tb, sem)
            cd = pltpu.async_copy(dt.at[dib], db, sem)
            cs = pltpu.async_copy(st.at[sib], sb, sem)
            cx.wait()
            ct.wait()
            cd.wait()
            cs.wait()

            def row_body(i, c):
                for u in range(D // LANES):
                    sl = pl.ds(u * LANES, LANES)
                    xb[i, sl] = xb[i, sl] + tb[i, sl] + db[i, sl] + sb[i, sl]
                return c

            lax.fori_loop(0, BLK, row_body, 0)
            pltpu.sync_copy(xb, out.at[rows])
            return carry

        lax.fori_loop(0, nblk, block_body, 0)

    return sc_call


def kernel(x, t_hour, t_day, spatial_indexs, time_table, day_table,
           spatial_table):
    B, T, N, D = x.shape
    R = B * T * N
    rows_per_w = R // NW
    assert rows_per_w * NW == R and rows_per_w % BLK == 0

    xf = x.reshape(R, D)
    hidx = t_hour.reshape(R).astype(jnp.int32)
    didx = t_day.reshape(R).astype(jnp.int32)
    sidx = jnp.tile(spatial_indexs.astype(jnp.int32), B * T)

    out = _make_sc_call(R, D, rows_per_w)(
        xf, hidx, didx, sidx, time_table, day_table, spatial_table)
    return out.reshape(B, T, N, D)
